# baseline (speedup 1.0000x reference)
import functools

import jax
import jax.numpy as jnp
from jax import lax
from jax.experimental import pallas as pl
from jax.experimental.pallas import tpu as pltpu
from jax.experimental.pallas import tpu_sc as plsc

_VOCAB = 100000
_SEQ = 2048
_BATCH = 4
_D = 128
_ROWS = _BATCH * _SEQ
_NC = 2
_NS = 16
_NW = _NC * _NS
_PPW = _SEQ // _NW
_CH = _PPW
_NCH = _BATCH
_L = 16


def _emb_body(ids_hbm, pos_hbm, tab_hbm, out_hbm, idx_v, rows_v,
              psems, gsems, osems):
    wid = lax.axis_index("s") * _NC + lax.axis_index("c")
    pos_base = wid * _PPW

    pdescs = [
        pltpu.async_copy(pos_hbm.at[pl.ds(pos_base, _PPW)], rows_v.at[j],
                         psems.at[j])
        for j in range(_NCH)
    ]
    for j in range(_NCH):
        row0 = j * _SEQ + pos_base
        pltpu.sync_copy(ids_hbm.at[pl.ds(row0, _CH)], idx_v.at[j])

    gdescs = []
    for j in range(_NCH):
        pdescs[j].wait()
        gdescs.append(
            pltpu.async_copy(tab_hbm.at[idx_v.at[j]], rows_v.at[j],
                             gsems.at[j], add=True))

    odescs = []
    for j in range(_NCH):
        gdescs[j].wait()
        odescs.append(
            pltpu.async_copy(rows_v.at[j],
                             out_hbm.at[pl.ds(j * _SEQ + pos_base, _CH)],
                             osems.at[j]))
    for d in odescs:
        d.wait()


@jax.jit
def _emb_call(ids_flat, token_embedding, position_embedding):
    mesh = plsc.VectorSubcoreMesh(core_axis_name="c", subcore_axis_name="s")
    run = pl.kernel(
        _emb_body,
        out_type=jax.ShapeDtypeStruct((_ROWS, _D), jnp.float32),
        mesh=mesh,
        scratch_types=[
            pltpu.VMEM((_NCH, _CH), jnp.int32),
            pltpu.VMEM((_NCH, _CH, _D), jnp.float32),
            pltpu.SemaphoreType.DMA((_NCH,)),
            pltpu.SemaphoreType.DMA((_NCH,)),
            pltpu.SemaphoreType.DMA((_NCH,)),
        ],
    )
    return run(ids_flat, position_embedding, token_embedding)


def kernel(token_ids, token_embedding, position_embedding):
    ids_flat = jnp.reshape(token_ids, (_ROWS,)).astype(jnp.int32)
    out = _emb_call(ids_flat, token_embedding, position_embedding)
    return jnp.reshape(out, (_BATCH, _SEQ, _D))


# speedup vs baseline: 1.0390x; 1.0390x over previous
import functools

import jax
import jax.numpy as jnp
from jax import lax
from jax.experimental import pallas as pl
from jax.experimental.pallas import tpu as pltpu
from jax.experimental.pallas import tpu_sc as plsc

_VOCAB = 100000
_SEQ = 2048
_BATCH = 4
_D = 128
_ROWS = _BATCH * _SEQ
_NC = 2
_NS = 16
_NW = _NC * _NS
_PPW = _SEQ // _NW
_CH = _PPW
_NCH = _BATCH
_L = 16


def _emb_body(ids_hbm, pos_hbm, tab_hbm, out_hbm, idx_v, rows_v, pos_v,
              psem, gsems, osems):
    wid = lax.axis_index("s") * _NC + lax.axis_index("c")
    pos_base = wid * _PPW

    pdesc = pltpu.async_copy(pos_hbm.at[pl.ds(pos_base, _PPW)], pos_v, psem)

    gdescs = []
    for j in range(_NCH):
        row0 = j * _SEQ + pos_base
        pltpu.sync_copy(ids_hbm.at[pl.ds(row0, _CH)], idx_v.at[j])
        gdescs.append(
            pltpu.async_copy(tab_hbm.at[idx_v.at[j]], rows_v.at[j],
                             gsems.at[j]))
    pdesc.wait()
    for d in gdescs:
        d.wait()

    def add_row(r, carry):
        for c in range(_D // _L):
            sl = pl.ds(c * _L, _L)
            pv = pos_v[r, sl]
            for j in range(_NCH):
                plsc.addupdate(rows_v.at[j, r, sl], pv)
        return carry
    lax.fori_loop(0, _CH, add_row, 0)

    odescs = [
        pltpu.async_copy(rows_v.at[j],
                         out_hbm.at[pl.ds(j * _SEQ + pos_base, _CH)],
                         osems.at[j])
        for j in range(_NCH)
    ]
    for d in odescs:
        d.wait()


@jax.jit
def _emb_call(ids_flat, token_embedding, position_embedding):
    mesh = plsc.VectorSubcoreMesh(core_axis_name="c", subcore_axis_name="s")
    run = pl.kernel(
        _emb_body,
        out_type=jax.ShapeDtypeStruct((_ROWS, _D), jnp.float32),
        mesh=mesh,
        scratch_types=[
            pltpu.VMEM((_NCH, _CH), jnp.int32),
            pltpu.VMEM((_NCH, _CH, _D), jnp.float32),
            pltpu.VMEM((_PPW, _D), jnp.float32),
            pltpu.SemaphoreType.DMA,
            pltpu.SemaphoreType.DMA((_NCH,)),
            pltpu.SemaphoreType.DMA((_NCH,)),
        ],
    )
    return run(ids_flat, position_embedding, token_embedding)


def kernel(token_ids, token_embedding, position_embedding):
    ids_flat = jnp.reshape(token_ids, (_ROWS,)).astype(jnp.int32)
    out = _emb_call(ids_flat, token_embedding, position_embedding)
    return jnp.reshape(out, (_BATCH, _SEQ, _D))


# speedup vs baseline: 1.0777x; 1.0372x over previous
import functools

import jax
import jax.numpy as jnp
from jax import lax
from jax.experimental import pallas as pl
from jax.experimental.pallas import tpu as pltpu
from jax.experimental.pallas import tpu_sc as plsc

_VOCAB = 100000
_SEQ = 2048
_BATCH = 4
_D = 128
_ROWS = _BATCH * _SEQ
_NC = 2
_NS = 16
_NW = _NC * _NS
_PPW = _SEQ // _NW
_CH = _PPW
_NCH = _BATCH
_L = 16


def _emb_body(ids_hbm, pos_hbm, tab_hbm, out_hbm, idx_v, rows_v, pos_v,
              psem, gsems, osems):
    wid = lax.axis_index("s") * _NC + lax.axis_index("c")
    pos_base = wid * _PPW

    pdesc = pltpu.async_copy(pos_hbm.at[pl.ds(pos_base, _PPW)], pos_v, psem)

    gdescs = []
    for j in range(_NCH):
        row0 = j * _SEQ + pos_base
        pltpu.sync_copy(ids_hbm.at[pl.ds(row0, _CH)], idx_v.at[j])
        gdescs.append(
            pltpu.async_copy(tab_hbm.at[idx_v.at[j]], rows_v.at[j],
                             gsems.at[j]))
    pdesc.wait()

    odescs = []
    for p in range(_NCH // 2):
        js = (2 * p, 2 * p + 1)
        for j in js:
            gdescs[j].wait()

        def add_row(r, carry, js=js):
            for c in range(_D // _L):
                sl = pl.ds(c * _L, _L)
                pv = pos_v[r, sl]
                for j in js:
                    plsc.addupdate(rows_v.at[j, r, sl], pv)
            return carry
        lax.fori_loop(0, _CH, add_row, 0)
        for j in js:
            odescs.append(
                pltpu.async_copy(rows_v.at[j],
                                 out_hbm.at[pl.ds(j * _SEQ + pos_base, _CH)],
                                 osems.at[j]))
    for d in odescs:
        d.wait()


@jax.jit
def _emb_call(ids_flat, token_embedding, position_embedding):
    mesh = plsc.VectorSubcoreMesh(core_axis_name="c", subcore_axis_name="s")
    run = pl.kernel(
        _emb_body,
        out_type=jax.ShapeDtypeStruct((_ROWS, _D), jnp.float32),
        mesh=mesh,
        scratch_types=[
            pltpu.VMEM((_NCH, _CH), jnp.int32),
            pltpu.VMEM((_NCH, _CH, _D), jnp.float32),
            pltpu.VMEM((_PPW, _D), jnp.float32),
            pltpu.SemaphoreType.DMA,
            pltpu.SemaphoreType.DMA((_NCH,)),
            pltpu.SemaphoreType.DMA((_NCH,)),
        ],
    )
    return run(ids_flat, position_embedding, token_embedding)


def kernel(token_ids, token_embedding, position_embedding):
    ids_flat = jnp.reshape(token_ids, (_ROWS,)).astype(jnp.int32)
    out = _emb_call(ids_flat, token_embedding, position_embedding)
    return jnp.reshape(out, (_BATCH, _SEQ, _D))
